# baseline (device time: 132987 ns/iter reference)
import jax
import jax.numpy as jnp
from jax import lax
from jax.experimental import pallas as pl
from jax.experimental.pallas import tpu as pltpu

N_DEV = 4
B, S, H, Dh, Dr = 4, 256, 32, 128, 64
D = 4096
DC = 512
DCS = DC // N_DEV
HL = H // N_DEV
HD = HL * Dh
HR = HL * Dr
BS = B * S

_MESH = pl.DeviceIdType.MESH
F32 = jnp.float32
BF16 = jnp.bfloat16
MB = 1024 * 1024


def _gather_body(x_ref, wdkv_ref, wuk_ref, wuv_ref, wq_hbm, wqr_hbm,
                 c_out, wuk_out, wuv_out, q_out, qr_out,
                 ukbf, uvbf, wq_buf, wqr_buf,
                 c_comm, uk_comm, uv_comm,
                 wq_sem, wqr_sem, c_ss, c_rs,
                 uk_ss, uk_rs, uv_ss, uv_rs):
    my = lax.axis_index("i")
    right = lax.rem(my + 1, N_DEV)

    wq_load = pltpu.make_async_copy(
        wq_hbm.at[:, pl.ds(my * HD, HD)], wq_buf, wq_sem)
    wq_load.start()
    wqr_load = pltpu.make_async_copy(
        wqr_hbm.at[:, pl.ds(my * HR, HR)], wqr_buf, wqr_sem)
    wqr_load.start()

    barrier = pltpu.get_barrier_semaphore()
    for d in range(1, N_DEV):
        pl.semaphore_signal(barrier, inc=1,
                            device_id=(lax.rem(my + d, N_DEV),),
                            device_id_type=_MESH)
    pl.semaphore_wait(barrier, N_DEV - 1)

    ukbf[...] = wuk_ref[...].astype(BF16)
    uvbf[...] = wuv_ref[...].astype(BF16)

    sends = []
    for d in range(1, N_DEV):
        peer = lax.rem(my + d, N_DEV)
        colp = peer * HD
        for src_full, buf, ss, rs in ((ukbf, uk_comm, uk_ss, uk_rs),
                                      (uvbf, uv_comm, uv_ss, uv_rs)):
            r = pltpu.make_async_remote_copy(
                src_ref=src_full.at[:, pl.ds(colp, HD)],
                dst_ref=buf.at[d],
                send_sem=ss.at[d], recv_sem=rs.at[d],
                device_id=(peer,), device_id_type=_MESH)
            r.start()
            sends.append(r)

    col = my * HD
    uk_comm[0] = ukbf[:, pl.ds(col, HD)]
    uv_comm[0] = uvbf[:, pl.ds(col, HD)]

    x = x_ref[...]
    c_comm[0] = jnp.dot(x, wdkv_ref[...].astype(BF16),
                        preferred_element_type=F32).astype(BF16)

    hops = []
    for h in range(N_DEV - 1):
        r = pltpu.make_async_remote_copy(
            src_ref=c_comm.at[h], dst_ref=c_comm.at[h + 1],
            send_sem=c_ss.at[h], recv_sem=c_rs.at[h + 1],
            device_id=(right,), device_id_type=_MESH)
        hops.append(r)

    hops[0].start()
    wq_load.wait()
    q_out[:, :HD // 2] = jnp.dot(
        x, wq_buf[:, :HD // 2].astype(BF16),
        preferred_element_type=F32).astype(BF16)
    hops[0].wait()
    hops[1].start()
    q_out[:, HD // 2:] = jnp.dot(
        x, wq_buf[:, HD // 2:].astype(BF16),
        preferred_element_type=F32).astype(BF16)
    hops[1].wait()
    hops[2].start()
    wqr_load.wait()
    qr_out[...] = jnp.dot(x, wqr_buf[...].astype(BF16),
                          preferred_element_type=F32).astype(BF16)
    hops[2].wait()

    for r in sends:
        r.wait_send()
    for r in sends:
        r.wait_recv()

    for s in range(N_DEV):
        origin = lax.rem(my - s + N_DEV, N_DEV)
        c_out[:, pl.ds(origin * DCS, DCS)] = c_comm[s]
        wuk_out[pl.ds(origin * DCS, DCS), :] = uk_comm[s]
        wuv_out[pl.ds(origin * DCS, DCS), :] = uv_comm[s]


def _gather(x_bf, wdkv32, wuk32, wuv32, wq32, wqr32):
    return pl.pallas_call(
        _gather_body,
        out_shape=[
            jax.ShapeDtypeStruct((BS, DC), BF16),
            jax.ShapeDtypeStruct((DC, HD), BF16),
            jax.ShapeDtypeStruct((DC, HD), BF16),
            jax.ShapeDtypeStruct((BS, HD), BF16),
            jax.ShapeDtypeStruct((BS, HR), BF16),
        ],
        in_specs=[pl.BlockSpec(memory_space=pltpu.VMEM)] * 4
        + [pl.BlockSpec(memory_space=pl.ANY)] * 2,
        out_specs=[pl.BlockSpec(memory_space=pltpu.VMEM)] * 5,
        scratch_shapes=[
            pltpu.VMEM((DCS, D), BF16),
            pltpu.VMEM((DCS, D), BF16),
            pltpu.VMEM((D, HD), F32),
            pltpu.VMEM((D, HR), F32),
            pltpu.VMEM((N_DEV, BS, DCS), BF16),
            pltpu.VMEM((N_DEV, DCS, HD), BF16),
            pltpu.VMEM((N_DEV, DCS, HD), BF16),
            pltpu.SemaphoreType.DMA,
            pltpu.SemaphoreType.DMA,
            pltpu.SemaphoreType.DMA((N_DEV,)),
            pltpu.SemaphoreType.DMA((N_DEV,)),
            pltpu.SemaphoreType.DMA((N_DEV,)),
            pltpu.SemaphoreType.DMA((N_DEV,)),
            pltpu.SemaphoreType.DMA((N_DEV,)),
            pltpu.SemaphoreType.DMA((N_DEV,)),
        ],
        compiler_params=pltpu.CompilerParams(
            collective_id=0, vmem_limit_bytes=62 * MB),
    )(x_bf, wdkv32, wuk32, wuv32, wq32, wqr32)


def _attn_body(c_ref, wuk_ref, wuv_ref, q_ref, qr_ref, kr_ref,
               o_ref, k_s, v_s):
    c = c_ref[...]
    k_s[...] = jnp.dot(c, wuk_ref[...], preferred_element_type=F32).astype(BF16)
    v_s[...] = jnp.dot(c, wuv_ref[...], preferred_element_type=F32).astype(BF16)

    scale = (Dh + Dr) ** -0.5
    for b in range(B):
        r0 = b * S
        kr_b = kr_ref[r0:r0 + S, :]
        for h in range(HL):
            qh = q_ref[r0:r0 + S, h * Dh:(h + 1) * Dh]
            qrh = qr_ref[r0:r0 + S, h * Dr:(h + 1) * Dr]
            kh = k_s[r0:r0 + S, h * Dh:(h + 1) * Dh]
            vh = v_s[r0:r0 + S, h * Dh:(h + 1) * Dh]
            dn = (((1,), (1,)), ((), ()))
            sc = lax.dot_general(qh, kh, dn, preferred_element_type=F32)
            sc += lax.dot_general(qrh, kr_b, dn, preferred_element_type=F32)
            p = jnp.exp(sc * scale)
            p /= jnp.sum(p, axis=-1, keepdims=True)
            o = jnp.dot(p.astype(BF16), vh, preferred_element_type=F32)
            o_ref[r0:r0 + S, h * Dh:(h + 1) * Dh] = o.astype(BF16)


def _attn(c_full, wuk_my, wuv_my, q, qr, kr):
    return pl.pallas_call(
        _attn_body,
        out_shape=jax.ShapeDtypeStruct((BS, HD), BF16),
        in_specs=[pl.BlockSpec(memory_space=pltpu.VMEM)] * 6,
        out_specs=pl.BlockSpec(memory_space=pltpu.VMEM),
        scratch_shapes=[
            pltpu.VMEM((BS, HD), BF16),
            pltpu.VMEM((BS, HD), BF16),
        ],
        compiler_params=pltpu.CompilerParams(vmem_limit_bytes=62 * MB),
    )(c_full, wuk_my, wuv_my, q, qr, kr)


NCB = 2
CW = D // NCB
HH = HD // 2


def _out_body(o_ref, wo_hbm, out_ref, commR, commL, wo_buf,
              ssR, rsR, ssL, rsL, load_sems):
    my = lax.axis_index("i")
    left = lax.rem(my + N_DEV - 1, N_DEV)
    right = lax.rem(my + 1, N_DEV)

    def load(i):
        h, is_l = i // 2, i % 2
        if is_l:
            row = lax.rem(my + h, N_DEV) * HD + HH
        else:
            row = lax.rem(my - h + N_DEV, N_DEV) * HD
        cp = pltpu.make_async_copy(
            wo_hbm.at[pl.ds(row, HH), :],
            wo_buf.at[i % 4], load_sems.at[i % 4])
        cp.start()
        return cp

    loads = [load(0), load(1), load(2), load(3)]

    barrier = pltpu.get_barrier_semaphore()
    for nbr in (left, right):
        pl.semaphore_signal(barrier, inc=1, device_id=(nbr,),
                            device_id_type=_MESH)
    pl.semaphore_wait(barrier, 2)

    commR[0] = o_ref[:, :HH]
    commL[0] = o_ref[:, HH:]
    for h in range(N_DEV):
        hops = []
        if h < N_DEV - 1:
            for buf, ss, rs, tgt in ((commR, ssR, rsR, right),
                                     (commL, ssL, rsL, left)):
                r = pltpu.make_async_remote_copy(
                    src_ref=buf.at[h], dst_ref=buf.at[h + 1],
                    send_sem=ss.at[h], recv_sem=rs.at[h + 1],
                    device_id=(tgt,), device_id_type=_MESH)
                r.start()
                hops.append(r)
        loads[2 * h].wait()
        loads[2 * h + 1].wait()
        cr = commR[h]
        cl = commL[h]
        for j in range(NCB):
            partR = jnp.dot(
                cr, wo_buf[2 * h % 4, :, j * CW:(j + 1) * CW].astype(BF16),
                preferred_element_type=F32)
            partL = jnp.dot(
                cl, wo_buf[(2 * h + 1) % 4, :, j * CW:(j + 1) * CW].astype(BF16),
                preferred_element_type=F32)
            part = (partR + partL).reshape(B, S, CW)
            if h == 0:
                out_ref[:, :, j * CW:(j + 1) * CW] = part.astype(BF16)
            else:
                prev = out_ref[:, :, j * CW:(j + 1) * CW]
                out_ref[:, :, j * CW:(j + 1) * CW] = (prev + part).astype(BF16)
        if h + 2 < N_DEV:
            loads.append(load(2 * h + 4))
            loads.append(load(2 * h + 5))
        for r in hops:
            r.wait()


def _out_proj(o_my, wo32):
    return pl.pallas_call(
        _out_body,
        out_shape=jax.ShapeDtypeStruct((B, S, D), BF16),
        in_specs=[
            pl.BlockSpec(memory_space=pltpu.VMEM),
            pl.BlockSpec(memory_space=pl.ANY),
        ],
        out_specs=pl.BlockSpec(memory_space=pltpu.VMEM),
        scratch_shapes=[
            pltpu.VMEM((N_DEV, BS, HH), BF16),
            pltpu.VMEM((N_DEV, BS, HH), BF16),
            pltpu.VMEM((4, HH, D), F32),
            pltpu.SemaphoreType.DMA((N_DEV,)),
            pltpu.SemaphoreType.DMA((N_DEV,)),
            pltpu.SemaphoreType.DMA((N_DEV,)),
            pltpu.SemaphoreType.DMA((N_DEV,)),
            pltpu.SemaphoreType.DMA((4,)),
        ],
        compiler_params=pltpu.CompilerParams(
            collective_id=1, vmem_limit_bytes=62 * MB),
    )(o_my, wo32)


def kernel(x, Wdkv, Wuk, Wuv, Wq, Wqr, Wkr, Wo):
    x_bf = x.reshape(BS, D).astype(BF16)
    kr = jnp.dot(x_bf, Wkr.astype(BF16),
                 preferred_element_type=F32).astype(BF16)
    c_full, wuk_my, wuv_my, q, qr = _gather(x_bf, Wdkv, Wuk, Wuv, Wq, Wqr)
    o_my = _attn(c_full, wuk_my, wuv_my, q, qr, kr)
    return _out_proj(o_my, Wo)


# device time: 132826 ns/iter; 1.0012x vs baseline; 1.0012x over previous
import jax
import jax.numpy as jnp
from jax import lax
from jax.experimental import pallas as pl
from jax.experimental.pallas import tpu as pltpu

N_DEV = 4
B, S, H, Dh, Dr = 4, 256, 32, 128, 64
D = 4096
DC = 512
DCS = DC // N_DEV
HL = H // N_DEV
HD = HL * Dh
HR = HL * Dr
BS = B * S

_MESH = pl.DeviceIdType.MESH
F32 = jnp.float32
BF16 = jnp.bfloat16
MB = 1024 * 1024


def _gather_body(x_ref, wdkv_ref, wuk_ref, wuv_ref, wq_hbm, wqr_hbm,
                 c_out, wuk_out, wuv_out, q_out, qr_out,
                 ukbf, uvbf, wq_buf, wqr_buf,
                 c_comm, uk_comm, uv_comm,
                 wq_sem, wqr_sem, c_ss, c_rs,
                 uk_ss, uk_rs, uv_ss, uv_rs):
    my = lax.axis_index("i")
    right = lax.rem(my + 1, N_DEV)

    wq_load = pltpu.make_async_copy(
        wq_hbm.at[:, pl.ds(my * HD, HD)], wq_buf, wq_sem)
    wq_load.start()
    wqr_load = pltpu.make_async_copy(
        wqr_hbm.at[:, pl.ds(my * HR, HR)], wqr_buf, wqr_sem)
    wqr_load.start()

    barrier = pltpu.get_barrier_semaphore()
    for d in range(1, N_DEV):
        pl.semaphore_signal(barrier, inc=1,
                            device_id=(lax.rem(my + d, N_DEV),),
                            device_id_type=_MESH)
    pl.semaphore_wait(barrier, N_DEV - 1)

    ukbf[...] = wuk_ref[...].astype(BF16)
    uvbf[...] = wuv_ref[...].astype(BF16)

    sends = []
    for d in range(1, N_DEV):
        peer = lax.rem(my + d, N_DEV)
        colp = peer * HD
        for src_full, buf, ss, rs in ((ukbf, uk_comm, uk_ss, uk_rs),
                                      (uvbf, uv_comm, uv_ss, uv_rs)):
            r = pltpu.make_async_remote_copy(
                src_ref=src_full.at[:, pl.ds(colp, HD)],
                dst_ref=buf.at[d],
                send_sem=ss.at[d], recv_sem=rs.at[d],
                device_id=(peer,), device_id_type=_MESH)
            r.start()
            sends.append(r)

    col = my * HD
    uk_comm[0] = ukbf[:, pl.ds(col, HD)]
    uv_comm[0] = uvbf[:, pl.ds(col, HD)]

    x = x_ref[...]
    c_comm[0] = jnp.dot(x, wdkv_ref[...].astype(BF16),
                        preferred_element_type=F32).astype(BF16)

    hops = []
    for h in range(N_DEV - 1):
        r = pltpu.make_async_remote_copy(
            src_ref=c_comm.at[h], dst_ref=c_comm.at[h + 1],
            send_sem=c_ss.at[h], recv_sem=c_rs.at[h + 1],
            device_id=(right,), device_id_type=_MESH)
        hops.append(r)

    hops[0].start()
    wq_load.wait()
    q_out[:, :HD // 2] = jnp.dot(
        x, wq_buf[:, :HD // 2].astype(BF16),
        preferred_element_type=F32).astype(BF16)
    hops[0].wait()
    hops[1].start()
    q_out[:, HD // 2:] = jnp.dot(
        x, wq_buf[:, HD // 2:].astype(BF16),
        preferred_element_type=F32).astype(BF16)
    hops[1].wait()
    hops[2].start()
    wqr_load.wait()
    qr_out[...] = jnp.dot(x, wqr_buf[...].astype(BF16),
                          preferred_element_type=F32).astype(BF16)
    hops[2].wait()

    for r in sends:
        r.wait_send()
    for r in sends:
        r.wait_recv()

    for s in range(N_DEV):
        origin = lax.rem(my - s + N_DEV, N_DEV)
        c_out[:, pl.ds(origin * DCS, DCS)] = c_comm[s]
        wuk_out[pl.ds(origin * DCS, DCS), :] = uk_comm[s]
        wuv_out[pl.ds(origin * DCS, DCS), :] = uv_comm[s]


def _gather(x_bf, wdkv32, wuk32, wuv32, wq32, wqr32):
    return pl.pallas_call(
        _gather_body,
        out_shape=[
            jax.ShapeDtypeStruct((BS, DC), BF16),
            jax.ShapeDtypeStruct((DC, HD), BF16),
            jax.ShapeDtypeStruct((DC, HD), BF16),
            jax.ShapeDtypeStruct((BS, HD), BF16),
            jax.ShapeDtypeStruct((BS, HR), BF16),
        ],
        in_specs=[pl.BlockSpec(memory_space=pltpu.VMEM)] * 4
        + [pl.BlockSpec(memory_space=pl.ANY)] * 2,
        out_specs=[pl.BlockSpec(memory_space=pltpu.VMEM)] * 5,
        scratch_shapes=[
            pltpu.VMEM((DCS, D), BF16),
            pltpu.VMEM((DCS, D), BF16),
            pltpu.VMEM((D, HD), F32),
            pltpu.VMEM((D, HR), F32),
            pltpu.VMEM((N_DEV, BS, DCS), BF16),
            pltpu.VMEM((N_DEV, DCS, HD), BF16),
            pltpu.VMEM((N_DEV, DCS, HD), BF16),
            pltpu.SemaphoreType.DMA,
            pltpu.SemaphoreType.DMA,
            pltpu.SemaphoreType.DMA((N_DEV,)),
            pltpu.SemaphoreType.DMA((N_DEV,)),
            pltpu.SemaphoreType.DMA((N_DEV,)),
            pltpu.SemaphoreType.DMA((N_DEV,)),
            pltpu.SemaphoreType.DMA((N_DEV,)),
            pltpu.SemaphoreType.DMA((N_DEV,)),
        ],
        compiler_params=pltpu.CompilerParams(
            collective_id=0, vmem_limit_bytes=62 * MB),
    )(x_bf, wdkv32, wuk32, wuv32, wq32, wqr32)


def _attn_body(c_ref, wuk_ref, wuv_ref, q_ref, qr_ref, kr_ref,
               o_ref, k_s, v_s):
    c = c_ref[...]
    k_s[...] = jnp.dot(c, wuk_ref[...], preferred_element_type=F32).astype(BF16)
    v_s[...] = jnp.dot(c, wuv_ref[...], preferred_element_type=F32).astype(BF16)

    scale = (Dh + Dr) ** -0.5
    kr_b = kr_ref[...]
    for h in range(HL):
        qh = q_ref[:, h * Dh:(h + 1) * Dh]
        qrh = qr_ref[:, h * Dr:(h + 1) * Dr]
        kh = k_s[:, h * Dh:(h + 1) * Dh]
        vh = v_s[:, h * Dh:(h + 1) * Dh]
        dn = (((1,), (1,)), ((), ()))
        sc = lax.dot_general(qh, kh, dn, preferred_element_type=F32)
        sc += lax.dot_general(qrh, kr_b, dn, preferred_element_type=F32)
        p = jnp.exp(sc * scale)
        p /= jnp.sum(p, axis=-1, keepdims=True)
        o = jnp.dot(p.astype(BF16), vh, preferred_element_type=F32)
        o_ref[:, h * Dh:(h + 1) * Dh] = o.astype(BF16)


def _attn(c_full, wuk_my, wuv_my, q, qr, kr):
    return pl.pallas_call(
        _attn_body,
        grid=(B,),
        out_shape=jax.ShapeDtypeStruct((BS, HD), BF16),
        in_specs=[
            pl.BlockSpec((S, DC), lambda b: (b, 0)),
            pl.BlockSpec((DC, HD), lambda b: (0, 0)),
            pl.BlockSpec((DC, HD), lambda b: (0, 0)),
            pl.BlockSpec((S, HD), lambda b: (b, 0)),
            pl.BlockSpec((S, HR), lambda b: (b, 0)),
            pl.BlockSpec((S, Dr), lambda b: (b, 0)),
        ],
        out_specs=pl.BlockSpec((S, HD), lambda b: (b, 0)),
        scratch_shapes=[
            pltpu.VMEM((S, HD), BF16),
            pltpu.VMEM((S, HD), BF16),
        ],
        compiler_params=pltpu.CompilerParams(vmem_limit_bytes=62 * MB),
    )(c_full, wuk_my, wuv_my, q, qr, kr)


NCB = 2
CW = D // NCB
HH = HD // 2


def _out_body(o_ref, wo_hbm, out_ref, commR, commL, wo_buf,
              ssR, rsR, ssL, rsL, load_sems):
    my = lax.axis_index("i")
    left = lax.rem(my + N_DEV - 1, N_DEV)
    right = lax.rem(my + 1, N_DEV)

    def load(i):
        h, is_l = i // 2, i % 2
        if is_l:
            row = lax.rem(my + h, N_DEV) * HD + HH
        else:
            row = lax.rem(my - h + N_DEV, N_DEV) * HD
        cp = pltpu.make_async_copy(
            wo_hbm.at[pl.ds(row, HH), :],
            wo_buf.at[i % 4], load_sems.at[i % 4])
        cp.start()
        return cp

    loads = [load(0), load(1), load(2), load(3)]

    barrier = pltpu.get_barrier_semaphore()
    for nbr in (left, right):
        pl.semaphore_signal(barrier, inc=1, device_id=(nbr,),
                            device_id_type=_MESH)
    pl.semaphore_wait(barrier, 2)

    commR[0] = o_ref[:, :HH]
    commL[0] = o_ref[:, HH:]
    for h in range(N_DEV):
        hops = []
        if h < N_DEV - 1:
            for buf, ss, rs, tgt in ((commR, ssR, rsR, right),
                                     (commL, ssL, rsL, left)):
                r = pltpu.make_async_remote_copy(
                    src_ref=buf.at[h], dst_ref=buf.at[h + 1],
                    send_sem=ss.at[h], recv_sem=rs.at[h + 1],
                    device_id=(tgt,), device_id_type=_MESH)
                r.start()
                hops.append(r)
        loads[2 * h].wait()
        loads[2 * h + 1].wait()
        cr = commR[h]
        cl = commL[h]
        for j in range(NCB):
            partR = jnp.dot(
                cr, wo_buf[2 * h % 4, :, j * CW:(j + 1) * CW].astype(BF16),
                preferred_element_type=F32)
            partL = jnp.dot(
                cl, wo_buf[(2 * h + 1) % 4, :, j * CW:(j + 1) * CW].astype(BF16),
                preferred_element_type=F32)
            part = (partR + partL).reshape(B, S, CW)
            if h == 0:
                out_ref[:, :, j * CW:(j + 1) * CW] = part.astype(BF16)
            else:
                prev = out_ref[:, :, j * CW:(j + 1) * CW]
                out_ref[:, :, j * CW:(j + 1) * CW] = (prev + part).astype(BF16)
        if h + 2 < N_DEV:
            loads.append(load(2 * h + 4))
            loads.append(load(2 * h + 5))
        for r in hops:
            r.wait()


def _out_proj(o_my, wo32):
    return pl.pallas_call(
        _out_body,
        out_shape=jax.ShapeDtypeStruct((B, S, D), BF16),
        in_specs=[
            pl.BlockSpec(memory_space=pltpu.VMEM),
            pl.BlockSpec(memory_space=pl.ANY),
        ],
        out_specs=pl.BlockSpec(memory_space=pltpu.VMEM),
        scratch_shapes=[
            pltpu.VMEM((N_DEV, BS, HH), BF16),
            pltpu.VMEM((N_DEV, BS, HH), BF16),
            pltpu.VMEM((4, HH, D), F32),
            pltpu.SemaphoreType.DMA((N_DEV,)),
            pltpu.SemaphoreType.DMA((N_DEV,)),
            pltpu.SemaphoreType.DMA((N_DEV,)),
            pltpu.SemaphoreType.DMA((N_DEV,)),
            pltpu.SemaphoreType.DMA((4,)),
        ],
        compiler_params=pltpu.CompilerParams(
            collective_id=1, vmem_limit_bytes=62 * MB),
    )(o_my, wo32)


def kernel(x, Wdkv, Wuk, Wuv, Wq, Wqr, Wkr, Wo):
    x_bf = x.reshape(BS, D).astype(BF16)
    kr = jnp.dot(x_bf, Wkr.astype(BF16),
                 preferred_element_type=F32).astype(BF16)
    c_full, wuk_my, wuv_my, q, qr = _gather(x_bf, Wdkv, Wuk, Wuv, Wq, Wqr)
    o_my = _attn(c_full, wuk_my, wuv_my, q, qr, kr)
    return _out_proj(o_my, Wo)
